# Initial kernel scaffold; baseline (speedup 1.0000x reference)
import jax
import jax.numpy as jnp
from jax.experimental import pallas as pl


def kernel(inputs, W):
    raise NotImplementedError("write your pallas kernel here")


# speedup vs baseline: 1.3575x; 1.3575x over previous
import functools

import jax
import jax.numpy as jnp
from jax import lax
from jax.experimental import pallas as pl
from jax.experimental.pallas import tpu as pltpu
from jax.experimental.pallas import tpu_sc as plsc

V = 8192
D = 256
B = 16
HW = 1024
M = B * HW

BC = 512
NCB = V // BC
BG = 4

NC, NS = 2, 16
NW = NC * NS
ROWS_PER_W = M // NW
CHUNK = 128
NCHUNK = ROWS_PER_W // CHUNK
HISTW = 128


def _argmin_body(x_ref, w_ref, idx_ref, minv_ref, rmin_ref, rarg_ref):
    c = pl.program_id(1)
    Wc = w_ref[...]
    wsq = jnp.sum(Wc * Wc, axis=1)
    for bb in range(BG):
        X = x_ref[bb]
        colsq = jnp.sum(X * X, axis=0)
        mm = lax.dot_general(
            Wc, X, (((1,), (0,)), ((), ())),
            preferred_element_type=jnp.float32,
            precision=lax.Precision.DEFAULT)
        dist = (colsq[None, :] + wsq[:, None]) - 2.0 * mm
        lmin = jnp.min(dist, axis=0)
        rows = lax.broadcasted_iota(jnp.int32, (BC, HW), 0)
        larg = jnp.min(jnp.where(dist == lmin[None, :], rows, BC),
                       axis=0) + c * BC

        @pl.when(c == 0)
        def _():
            rmin_ref[bb, :] = lmin
            rarg_ref[bb, :] = larg

        @pl.when(c > 0)
        def _():
            prev = rmin_ref[bb, :]
            better = lmin < prev
            rmin_ref[bb, :] = jnp.where(better, lmin, prev)
            rarg_ref[bb, :] = jnp.where(better, larg, rarg_ref[bb, :])

        @pl.when(c == NCB - 1)
        def _():
            idx_ref[bb, 0, :] = rarg_ref[bb, :]
            minv_ref[bb, 0, :] = rmin_ref[bb, :]


def _argmin_call(X3, W):
    return pl.pallas_call(
        _argmin_body,
        grid=(B // BG, NCB),
        in_specs=[
            pl.BlockSpec((BG, D, HW), lambda b, c: (b, 0, 0)),
            pl.BlockSpec((BC, D), lambda b, c: (c, 0)),
        ],
        out_specs=[
            pl.BlockSpec((BG, 1, HW), lambda b, c: (b, 0, 0)),
            pl.BlockSpec((BG, 1, HW), lambda b, c: (b, 0, 0)),
        ],
        out_shape=[
            jax.ShapeDtypeStruct((B, 1, HW), jnp.int32),
            jax.ShapeDtypeStruct((B, 1, HW), jnp.float32),
        ],
        scratch_shapes=[
            pltpu.VMEM((BG, HW), jnp.float32),
            pltpu.VMEM((BG, HW), jnp.int32),
        ],
        compiler_params=pltpu.CompilerParams(
            dimension_semantics=("arbitrary", "arbitrary")),
    )(X3, W)


def _sc_body(w_hbm, idx_hbm, ones_hbm, zeros_hbm,
             quant_out, counts_out,
             idx_c, rows_v, ones_v, hist_sh, sem):
    c = lax.axis_index("c")
    s = lax.axis_index("s")
    wid = s * NC + c

    @pl.when(s == 0)
    def _():
        pltpu.sync_copy(zeros_hbm, hist_sh)

    pltpu.sync_copy(ones_hbm, ones_v)
    plsc.subcore_barrier()

    for j in range(NCHUNK):
        base = pl.multiple_of(wid * ROWS_PER_W + j * CHUNK, CHUNK)
        pltpu.sync_copy(idx_hbm.at[pl.ds(base, CHUNK)], idx_c)
        pltpu.async_copy(w_hbm.at[idx_c], rows_v, sem).wait()
        pltpu.sync_copy(rows_v, quant_out.at[pl.ds(base, CHUNK)])
        pltpu.sync_copy(ones_v, hist_sh.at[idx_c], add=True)

    plsc.subcore_barrier()

    @pl.when(s == 0)
    def _():
        pltpu.sync_copy(hist_sh, counts_out.at[c])


@functools.lru_cache(maxsize=1)
def _sc_gather_hist():
    mesh = plsc.VectorSubcoreMesh(
        core_axis_name="c", subcore_axis_name="s",
        num_cores=NC, num_subcores=NS)
    return pl.kernel(
        _sc_body,
        out_type=(
            jax.ShapeDtypeStruct((M, D), jnp.float32),
            jax.ShapeDtypeStruct((NC, V, HISTW), jnp.float32),
        ),
        mesh=mesh,
        scratch_types=[
            pltpu.VMEM((CHUNK,), jnp.int32),
            pltpu.VMEM((CHUNK, D), jnp.float32),
            pltpu.VMEM((CHUNK, HISTW), jnp.float32),
            pltpu.VMEM_SHARED((V, HISTW), jnp.float32),
            pltpu.SemaphoreType.DMA,
        ],
    )


def _finalize_body(cnt_ref, minv_ref, loss_ref, perp_ref):
    cnt = cnt_ref[0, :, 0] + cnt_ref[1, :, 0]
    avg = cnt * (1.0 / M)
    ent = jnp.sum(avg * jnp.log(avg + 1e-10))
    perp_ref[0, 0] = jnp.exp(-ent)
    lsum = jnp.sum(minv_ref[...])
    loss_ref[0, 0] = 1.25 * lsum / (M * D)


def _finalize_call(counts, minv3):
    return pl.pallas_call(
        _finalize_body,
        out_specs=[
            pl.BlockSpec(memory_space=pltpu.SMEM),
            pl.BlockSpec(memory_space=pltpu.SMEM),
        ],
        out_shape=[
            jax.ShapeDtypeStruct((1, 1), jnp.float32),
            jax.ShapeDtypeStruct((1, 1), jnp.float32),
        ],
    )(counts, minv3)


@jax.jit
def kernel(inputs, W):
    X3 = inputs.reshape(B, D, HW)
    idx3, minv3 = _argmin_call(X3, W)
    idx_flat = idx3.reshape(M)
    ones = jnp.ones((CHUNK, HISTW), jnp.float32)
    zeros = jnp.zeros((V, HISTW), jnp.float32)
    quant, counts = _sc_gather_hist()(W, idx_flat, ones, zeros)
    loss2, perp2 = _finalize_call(counts, minv3)
    q = quant.reshape(B, 32, 32, D).transpose(0, 3, 1, 2)
    return q, loss2[0, 0], perp2[0, 0]


# speedup vs baseline: 1.3634x; 1.0043x over previous
import functools

import jax
import jax.numpy as jnp
from jax import lax
from jax.experimental import pallas as pl
from jax.experimental.pallas import tpu as pltpu
from jax.experimental.pallas import tpu_sc as plsc

V = 8192
D = 256
B = 16
HW = 1024
M = B * HW

BC = 512
NCB = V // BC
BG = 8

NC, NS = 2, 16
NW = NC * NS
ROWS_PER_W = M // NW
CHUNK = 128
NCHUNK = ROWS_PER_W // CHUNK
HISTW = 128


def _argmin_body(x_ref, w_ref, idx_ref, minv_ref, rmin_ref, rarg_ref):
    c = pl.program_id(1)
    Wc = w_ref[...]
    wsq = jnp.sum(Wc * Wc, axis=1)
    for bb in range(BG):
        X = x_ref[bb]
        colsq = jnp.sum(X * X, axis=0)
        mm = lax.dot_general(
            Wc, X, (((1,), (0,)), ((), ())),
            preferred_element_type=jnp.float32,
            precision=lax.Precision.DEFAULT)
        dist = (colsq[None, :] + wsq[:, None]) - 2.0 * mm
        lmin = jnp.min(dist, axis=0)
        rows = lax.broadcasted_iota(jnp.int32, (BC, HW), 0)
        larg = jnp.min(jnp.where(dist == lmin[None, :], rows, BC),
                       axis=0) + c * BC

        @pl.when(c == 0)
        def _():
            rmin_ref[bb, :] = lmin
            rarg_ref[bb, :] = larg

        @pl.when(c > 0)
        def _():
            prev = rmin_ref[bb, :]
            better = lmin < prev
            rmin_ref[bb, :] = jnp.where(better, lmin, prev)
            rarg_ref[bb, :] = jnp.where(better, larg, rarg_ref[bb, :])

        @pl.when(c == NCB - 1)
        def _():
            idx_ref[bb, 0, :] = rarg_ref[bb, :]
            minv_ref[bb, 0, :] = rmin_ref[bb, :]


def _argmin_call(X3, W):
    return pl.pallas_call(
        _argmin_body,
        grid=(B // BG, NCB),
        in_specs=[
            pl.BlockSpec((BG, D, HW), lambda b, c: (b, 0, 0)),
            pl.BlockSpec((BC, D), lambda b, c: (c, 0)),
        ],
        out_specs=[
            pl.BlockSpec((BG, 1, HW), lambda b, c: (b, 0, 0)),
            pl.BlockSpec((BG, 1, HW), lambda b, c: (b, 0, 0)),
        ],
        out_shape=[
            jax.ShapeDtypeStruct((B, 1, HW), jnp.int32),
            jax.ShapeDtypeStruct((B, 1, HW), jnp.float32),
        ],
        scratch_shapes=[
            pltpu.VMEM((BG, HW), jnp.float32),
            pltpu.VMEM((BG, HW), jnp.int32),
        ],
        compiler_params=pltpu.CompilerParams(
            dimension_semantics=("arbitrary", "arbitrary")),
    )(X3, W)


def _sc_body(w_hbm, idx_hbm, ones_hbm, zeros_hbm,
             quant_out, counts_out,
             idx_c, rows_v, ones_v, hist_sh, sem):
    c = lax.axis_index("c")
    s = lax.axis_index("s")
    wid = s * NC + c

    @pl.when(s == 0)
    def _():
        pltpu.sync_copy(zeros_hbm, hist_sh)

    pltpu.sync_copy(ones_hbm, ones_v)
    plsc.subcore_barrier()

    for j in range(NCHUNK):
        base = pl.multiple_of(wid * ROWS_PER_W + j * CHUNK, CHUNK)
        pltpu.sync_copy(idx_hbm.at[pl.ds(base, CHUNK)], idx_c)
        pltpu.async_copy(w_hbm.at[idx_c], rows_v, sem).wait()
        pltpu.sync_copy(rows_v, quant_out.at[pl.ds(base, CHUNK)])
        pltpu.sync_copy(ones_v, hist_sh.at[idx_c], add=True)

    plsc.subcore_barrier()

    @pl.when(s == 0)
    def _():
        pltpu.sync_copy(hist_sh, counts_out.at[c])


@functools.lru_cache(maxsize=1)
def _sc_gather_hist():
    mesh = plsc.VectorSubcoreMesh(
        core_axis_name="c", subcore_axis_name="s",
        num_cores=NC, num_subcores=NS)
    return pl.kernel(
        _sc_body,
        out_type=(
            jax.ShapeDtypeStruct((M, D), jnp.float32),
            jax.ShapeDtypeStruct((NC, V, HISTW), jnp.float32),
        ),
        mesh=mesh,
        scratch_types=[
            pltpu.VMEM((CHUNK,), jnp.int32),
            pltpu.VMEM((CHUNK, D), jnp.float32),
            pltpu.VMEM((CHUNK, HISTW), jnp.float32),
            pltpu.VMEM_SHARED((V, HISTW), jnp.float32),
            pltpu.SemaphoreType.DMA,
        ],
    )


def _finalize_body(cnt_ref, minv_ref, loss_ref, perp_ref):
    cnt = cnt_ref[0, :, 0] + cnt_ref[1, :, 0]
    avg = cnt * (1.0 / M)
    ent = jnp.sum(avg * jnp.log(avg + 1e-10))
    perp_ref[0, 0] = jnp.exp(-ent)
    lsum = jnp.sum(minv_ref[...])
    loss_ref[0, 0] = 1.25 * lsum / (M * D)


def _finalize_call(counts, minv3):
    return pl.pallas_call(
        _finalize_body,
        out_specs=[
            pl.BlockSpec(memory_space=pltpu.SMEM),
            pl.BlockSpec(memory_space=pltpu.SMEM),
        ],
        out_shape=[
            jax.ShapeDtypeStruct((1, 1), jnp.float32),
            jax.ShapeDtypeStruct((1, 1), jnp.float32),
        ],
    )(counts, minv3)


@jax.jit
def kernel(inputs, W):
    X3 = inputs.reshape(B, D, HW)
    idx3, minv3 = _argmin_call(X3, W)
    idx_flat = idx3.reshape(M)
    ones = jnp.ones((CHUNK, HISTW), jnp.float32)
    zeros = jnp.zeros((V, HISTW), jnp.float32)
    quant, counts = _sc_gather_hist()(W, idx_flat, ones, zeros)
    loss2, perp2 = _finalize_call(counts, minv3)
    q = quant.reshape(B, 32, 32, D).transpose(0, 3, 1, 2)
    return q, loss2[0, 0], perp2[0, 0]
